# Initial kernel scaffold; baseline (speedup 1.0000x reference)
import jax
import jax.numpy as jnp
from jax.experimental import pallas as pl


def kernel(x, y, emb_table, W, b):
    raise NotImplementedError("write your pallas kernel here")


# speedup vs baseline: 33.0595x; 33.0595x over previous
import functools

import jax
import jax.numpy as jnp
from jax import lax
from jax.experimental import pallas as pl
from jax.experimental.pallas import tpu as pltpu
from jax.experimental.pallas import tpu_sc as plsc

B, L, D, V = 4096, 128, 128, 1000
VP = 1024
NC, NS = 2, 16
NW = NC * NS
CHUNK = (B * L) // NW
LANES = 16


def _row_mean_body(emb_ref, out_ref):
    out_ref[...] = jnp.mean(emb_ref[...], axis=2)


def _logits_softmax_body(p_ref, w_ref, b_ref, out_ref):
    logits = lax.dot_general(
        p_ref[...], w_ref[...],
        (((1,), (1,)), ((), ())),
        preferred_element_type=jnp.float32,
    ) + b_ref[...]
    mx = jnp.max(logits, axis=1, keepdims=True)
    e = jnp.exp(logits - mx)
    out_ref[...] = e / jnp.sum(e, axis=1, keepdims=True)


@functools.lru_cache(maxsize=1)
def _make_sc_gather():
    mesh = plsc.VectorSubcoreMesh(core_axis_name="c", subcore_axis_name="s")

    @functools.partial(
        pl.kernel,
        mesh=mesh,
        out_type=jax.ShapeDtypeStruct((B * L,), jnp.float32),
        scratch_types=[
            pltpu.VMEM((CHUNK,), jnp.int32),
            pltpu.VMEM((CHUNK,), jnp.float32),
            pltpu.VMEM((VP,), jnp.float32),
        ],
        compiler_params=pltpu.CompilerParams(needs_layout_passes=False),
    )
    def _sc_gather(x_hbm, m_hbm, out_hbm, idx_v, pooled_v, m_v):
        wid = lax.axis_index("s") * NC + lax.axis_index("c")
        base = wid * CHUNK
        pltpu.sync_copy(m_hbm, m_v)
        pltpu.sync_copy(x_hbm.at[pl.ds(base, CHUNK)], idx_v)

        def body(i, carry):
            off = i * LANES
            idx = idx_v[pl.ds(off, LANES)]
            pooled_v[pl.ds(off, LANES)] = plsc.load_gather(m_v, [idx])
            return carry

        lax.fori_loop(0, CHUNK // LANES, body, 0)
        pltpu.sync_copy(pooled_v, out_hbm.at[pl.ds(base, CHUNK)])

    return _sc_gather


def kernel(x, y, emb_table, W, b):
    del y
    x = x.astype(jnp.int32)

    emb3 = jnp.pad(emb_table, ((0, VP - V), (0, 0))).reshape(VP // D, D, D)
    m2d = pl.pallas_call(
        _row_mean_body,
        out_shape=jax.ShapeDtypeStruct((VP // D, D), jnp.float32),
    )(emb3)
    m = m2d.reshape(VP)

    pooled = _make_sc_gather()(x.reshape(B * L), m).reshape(B, L)

    Wp = jnp.pad(W, ((0, VP - V), (0, 0)))
    bp = jnp.concatenate([b, jnp.full((VP - V,), -1e30, jnp.float32)])
    bp = bp.reshape(1, VP)
    BM = 512
    probs = pl.pallas_call(
        _logits_softmax_body,
        grid=(B // BM,),
        in_specs=[
            pl.BlockSpec((BM, D), lambda i: (i, 0)),
            pl.BlockSpec((VP, D), lambda i: (0, 0)),
            pl.BlockSpec((1, VP), lambda i: (0, 0)),
        ],
        out_specs=pl.BlockSpec((BM, VP), lambda i: (i, 0)),
        out_shape=jax.ShapeDtypeStruct((B, VP), jnp.float32),
    )(pooled, Wp, bp)
    return probs[:, :V]
